# baseline (speedup 1.0000x reference)
import functools

import jax
import jax.numpy as jnp
from jax import lax
from jax.experimental import pallas as pl
from jax.experimental.pallas import tpu as pltpu
from jax.experimental.pallas import tpu_sc as plsc

NUM_USERS = 6000
NUM_ITEMS = 4000
N_NODES = NUM_USERS + NUM_ITEMS
D = 128
N_LAYERS = 3
N_EDGES = 320000
BATCH = 4096

NC = 2
NS = 16
NW = NC * NS
CHUNK = 32
GCHUNK = 128
EDGES_PER_TILE = 10240
N_CHUNKS = EDGES_PER_TILE // CHUNK
NP = 10240
ROWS_PER_TILE = NP // NS

_mesh = plsc.VectorSubcoreMesh(
    core_axis_name="c", subcore_axis_name="s", num_cores=NC, num_subcores=NS)


NBUF = 8
LG = NBUF - 2
NGROUPS = N_CHUNKS // NBUF


def _prop_body(pref_hbm, cols4, rows4, vals4, zeros_hbm, out_hbm,
               cols_e, rows_e, vals_e, g0, g1, g2, g3, g4, g5, g6, g7, shared,
               esem, gs0, gs1, gs2, gs3, gs4, gs5, gs6, gs7,
               ss0, ss1, ss2, ss3, ss4, ss5, ss6, ss7):
    gbuf = [g0, g1, g2, g3, g4, g5, g6, g7]
    gsem = [gs0, gs1, gs2, gs3, gs4, gs5, gs6, gs7]
    ssem = [ss0, ss1, ss2, ss3, ss4, ss5, ss6, ss7]
    c = lax.axis_index("c")
    s = lax.axis_index("s")
    wid = c * NS + s

    def eissue(g, slot):
        pltpu.async_copy(cols4.at[wid, g], cols_e.at[slot], esem)
        pltpu.async_copy(rows4.at[wid, g], rows_e.at[slot], esem)
        pltpu.async_copy(vals4.at[wid, g], vals_e.at[slot], esem)

    def ewait(g, slot):
        pltpu.make_async_copy(cols4.at[wid, g], cols_e.at[slot], esem).wait()
        pltpu.make_async_copy(rows4.at[wid, g], rows_e.at[slot], esem).wait()
        pltpu.make_async_copy(vals4.at[wid, g], vals_e.at[slot], esem).wait()

    def gissue(slot, bslot, b):
        pltpu.async_copy(pref_hbm.at[cols_e.at[slot, bslot]], gbuf[b], gsem[b])

    def gwait(slot, bslot, b):
        pltpu.make_async_copy(pref_hbm.at[cols_e.at[slot, bslot]],
                              gbuf[b], gsem[b]).wait()

    def sissue(slot, bslot, b):
        pltpu.async_copy(gbuf[b], shared.at[rows_e.at[slot, bslot]], ssem[b],
                         add=True)

    def swait(slot, bslot, b):
        pltpu.make_async_copy(gbuf[b], shared.at[rows_e.at[slot, bslot]],
                              ssem[b]).wait()

    pltpu.sync_copy(zeros_hbm.at[pl.ds(s * ROWS_PER_TILE, ROWS_PER_TILE)],
                    shared.at[pl.ds(s * ROWS_PER_TILE, ROWS_PER_TILE)])
    eissue(0, 0)
    ewait(0, 0)
    for b in range(LG):
        gissue(0, b, b)
    plsc.subcore_barrier()

    def group_body(p, carry):
        slot = lax.rem(p, 3)
        slot1 = lax.rem(p + 1, 3)

        @pl.when(p + 1 < NGROUPS)
        def _():
            eissue(p + 1, slot1)

        for b in range(NBUF):
            i = NBUF * p + b
            if b == NBUF - LG:
                @pl.when(p + 1 < NGROUPS)
                def _():
                    ewait(p + 1, slot1)
            gwait(slot, b, b)
            bn = (b + LG) % NBUF

            @pl.when(i >= NBUF - LG)
            def _():
                swait(slot if b >= NBUF - LG else lax.rem(p + 2, 3), bn, bn)

            @pl.when(i + LG < N_CHUNKS)
            def _():
                gissue(slot if b < NBUF - LG else slot1, bn, bn)

            def grp_body(g, carry2):
                vvec = vals_e[slot, b, pl.ds(g * 16, 16)]
                for j in range(16):
                    r = g * 16 + j
                    v = jnp.broadcast_to(vvec[j], (16,))
                    for k in range(D // 16):
                        sl = pl.ds(k * 16, 16)
                        gbuf[b][r, sl] = gbuf[b][r, sl] * v
                return carry2
            lax.fori_loop(0, CHUNK // 16, grp_body, 0)

            sissue(slot, b, b)
        return carry
    lax.fori_loop(0, NGROUPS, group_body, 0)

    last_slot = (NGROUPS - 1) % 3
    for b in range(LG, NBUF):
        swait(last_slot, b, b)
    plsc.subcore_barrier()
    pltpu.sync_copy(shared.at[pl.ds(s * ROWS_PER_TILE, ROWS_PER_TILE)],
                    out_hbm.at[c, pl.ds(s * ROWS_PER_TILE, ROWS_PER_TILE)])


_sc_propagate = functools.partial(
    pl.kernel,
    out_type=jax.ShapeDtypeStruct((NC, NP, D), jnp.float32),
    mesh=_mesh,
    scratch_types=[
        pltpu.VMEM((3, NBUF, CHUNK), jnp.int32),
        pltpu.VMEM((3, NBUF, CHUNK), jnp.int32),
        pltpu.VMEM((3, NBUF, CHUNK), jnp.float32),
    ] + [pltpu.VMEM((CHUNK, D), jnp.float32)] * NBUF + [
        pltpu.VMEM_SHARED((NP, D), jnp.float32),
    ] + [pltpu.SemaphoreType.DMA] * (2 * NBUF + 1),
)(_prop_body)


def _gather_body(table_hbm, idx3_hbm, out_hbm, idx_v, buf_v, sem):
    c = lax.axis_index("c")
    s = lax.axis_index("s")
    wid = c * NS + s
    pltpu.sync_copy(idx3_hbm.at[wid], idx_v)
    for j in range(4):
        pltpu.async_copy(table_hbm.at[idx_v.at[j]], buf_v, sem).wait()
        pltpu.sync_copy(
            buf_v, out_hbm.at[pl.ds(wid * 4 * GCHUNK + j * GCHUNK, GCHUNK)])


_sc_gather = functools.partial(
    pl.kernel,
    out_type=jax.ShapeDtypeStruct((4 * BATCH, D), jnp.float32),
    mesh=_mesh,
    scratch_types=[
        pltpu.VMEM((4, GCHUNK), jnp.int32),
        pltpu.VMEM((GCHUNK, D), jnp.float32),
        pltpu.SemaphoreType.DMA,
    ],
)(_gather_body)


def _leaky_norm(p):
    p = jnp.where(p >= 0, p, 0.1 * p)
    n = jnp.sqrt(jnp.sum(p * p, axis=-1, keepdims=True))
    return p / jnp.maximum(n, 1e-12)


def _norm_body(x_ref, o_ref):
    o_ref[...] = _leaky_norm(x_ref[...])


def _tc_norm(x, rows_per_block):
    n_rows = x.shape[0]
    grid = n_rows // rows_per_block
    return pl.pallas_call(
        _norm_body,
        grid=(grid,),
        in_specs=[pl.BlockSpec((rows_per_block, D), lambda i: (i, 0))],
        out_specs=pl.BlockSpec((rows_per_block, D), lambda i: (i, 0)),
        out_shape=jax.ShapeDtypeStruct((n_rows, D), jnp.float32),
    )(x)


def _combine_body(p_ref, acc_ref, o_ref, acco_ref):
    nrm = _leaky_norm(p_ref[0] + p_ref[1])
    o_ref[...] = nrm
    acco_ref[...] = acc_ref[...] + nrm


def _tc_combine(parts, acc):
    rb = 1024
    grid = NP // rb
    return pl.pallas_call(
        _combine_body,
        grid=(grid,),
        in_specs=[
            pl.BlockSpec((NC, rb, D), lambda i: (0, i, 0)),
            pl.BlockSpec((rb, D), lambda i: (i, 0)),
        ],
        out_specs=[
            pl.BlockSpec((rb, D), lambda i: (i, 0)),
            pl.BlockSpec((rb, D), lambda i: (i, 0)),
        ],
        out_shape=[
            jax.ShapeDtypeStruct((NP, D), jnp.float32),
            jax.ShapeDtypeStruct((NP, D), jnp.float32),
        ],
    )(parts, acc)


def _dots_body(g_ref, a_ref, w_ref, s_ref):
    scale = 1.0 / ((N_LAYERS + 1) * (N_LAYERS + 1))
    u = g_ref[0]
    a_ref[...] = jax.nn.sigmoid(jnp.sum(u * g_ref[1], axis=-1) * scale)
    w_ref[...] = jax.nn.sigmoid(jnp.sum(u * g_ref[2], axis=-1) * scale)
    s_ref[...] = jax.nn.sigmoid(jnp.sum(u * g_ref[3], axis=-1) * scale)


def _tc_dots(g4):
    bb = 1024
    grid = BATCH // bb
    out = jax.ShapeDtypeStruct((BATCH,), jnp.float32)
    return pl.pallas_call(
        _dots_body,
        grid=(grid,),
        in_specs=[pl.BlockSpec((4, bb, D), lambda i: (0, i, 0))],
        out_specs=[pl.BlockSpec((bb,), lambda i: (i,))] * 3,
        out_shape=[out, out, out],
    )(g4)


def kernel(users, adjacent_items, weak_items, strong_items, edge_index,
           edge_values, user_preference, item_preference):
    nu = _tc_norm(user_preference, 1000)
    ni = _tc_norm(item_preference, 1000)
    pref = jnp.concatenate(
        [nu, ni, jnp.zeros((NP - N_NODES, D), jnp.float32)], axis=0)
    acc = pref

    rows = edge_index[0]
    cols = edge_index[1]
    pad = NW * EDGES_PER_TILE - N_EDGES
    eshape = (NW, NGROUPS, NBUF, CHUNK)
    spread = (jnp.arange(pad, dtype=jnp.int32) * 13) % N_NODES
    rows4 = jnp.concatenate([rows, spread])
    rows4 = rows4.reshape(eshape)
    cols4 = jnp.concatenate([cols, spread])
    cols4 = cols4.reshape(eshape)
    vals4 = jnp.concatenate([edge_values, jnp.zeros((pad,), jnp.float32)])
    vals4 = vals4.reshape(eshape)
    zeros = jnp.zeros((NP, D), jnp.float32)

    for _ in range(N_LAYERS):
        parts = _sc_propagate(pref, cols4, rows4, vals4, zeros)
        pref, acc = _tc_combine(parts, acc)

    idx = jnp.concatenate([
        users,
        adjacent_items + NUM_USERS,
        weak_items + NUM_USERS,
        strong_items + NUM_USERS,
    ]).reshape(NW, 4, GCHUNK)
    g = _sc_gather(acc, idx)
    return _tc_dots(g.reshape(4, BATCH, D))


# speedup vs baseline: 1.0171x; 1.0171x over previous
import functools

import jax
import jax.numpy as jnp
from jax import lax
from jax.experimental import pallas as pl
from jax.experimental.pallas import tpu as pltpu
from jax.experimental.pallas import tpu_sc as plsc

NUM_USERS = 6000
NUM_ITEMS = 4000
N_NODES = NUM_USERS + NUM_ITEMS
D = 128
N_LAYERS = 3
N_EDGES = 320000
BATCH = 4096

NC = 2
NS = 16
NW = NC * NS
CHUNK = 64
GCHUNK = 128
EDGES_PER_TILE = 10240
N_CHUNKS = EDGES_PER_TILE // CHUNK
NP = 10240
ROWS_PER_TILE = NP // NS

_mesh = plsc.VectorSubcoreMesh(
    core_axis_name="c", subcore_axis_name="s", num_cores=NC, num_subcores=NS)


NBUF = 4
LG = NBUF - 2
NGROUPS = N_CHUNKS // NBUF


def _prop_body(pref_hbm, cols4, rows4, vals4, zeros_hbm, out_hbm,
               cols_e, rows_e, vals_e, g0, g1, g2, g3, shared,
               esem, gs0, gs1, gs2, gs3, ss0, ss1, ss2, ss3):
    gbuf = [g0, g1, g2, g3]
    gsem = [gs0, gs1, gs2, gs3]
    ssem = [ss0, ss1, ss2, ss3]
    c = lax.axis_index("c")
    s = lax.axis_index("s")
    wid = c * NS + s

    def eissue(g, slot):
        pltpu.async_copy(cols4.at[wid, g], cols_e.at[slot], esem)
        pltpu.async_copy(rows4.at[wid, g], rows_e.at[slot], esem)
        pltpu.async_copy(vals4.at[wid, g], vals_e.at[slot], esem)

    def ewait(g, slot):
        pltpu.make_async_copy(cols4.at[wid, g], cols_e.at[slot], esem).wait()
        pltpu.make_async_copy(rows4.at[wid, g], rows_e.at[slot], esem).wait()
        pltpu.make_async_copy(vals4.at[wid, g], vals_e.at[slot], esem).wait()

    def gissue(slot, bslot, b):
        pltpu.async_copy(pref_hbm.at[cols_e.at[slot, bslot]], gbuf[b], gsem[b])

    def gwait(slot, bslot, b):
        pltpu.make_async_copy(pref_hbm.at[cols_e.at[slot, bslot]],
                              gbuf[b], gsem[b]).wait()

    def sissue(slot, bslot, b):
        pltpu.async_copy(gbuf[b], shared.at[rows_e.at[slot, bslot]], ssem[b],
                         add=True)

    def swait(slot, bslot, b):
        pltpu.make_async_copy(gbuf[b], shared.at[rows_e.at[slot, bslot]],
                              ssem[b]).wait()

    pltpu.sync_copy(zeros_hbm.at[pl.ds(s * ROWS_PER_TILE, ROWS_PER_TILE)],
                    shared.at[pl.ds(s * ROWS_PER_TILE, ROWS_PER_TILE)])
    eissue(0, 0)
    ewait(0, 0)
    for b in range(LG):
        gissue(0, b, b)
    plsc.subcore_barrier()

    def group_body(p, carry):
        slot = lax.rem(p, 3)
        slot1 = lax.rem(p + 1, 3)

        @pl.when(p + 1 < NGROUPS)
        def _():
            eissue(p + 1, slot1)

        for b in range(NBUF):
            i = NBUF * p + b
            if b == NBUF - LG:
                @pl.when(p + 1 < NGROUPS)
                def _():
                    ewait(p + 1, slot1)
            gwait(slot, b, b)
            bn = (b + LG) % NBUF

            @pl.when(i >= NBUF - LG)
            def _():
                swait(slot if b >= NBUF - LG else lax.rem(p + 2, 3), bn, bn)

            @pl.when(i + LG < N_CHUNKS)
            def _():
                gissue(slot if b + LG < NBUF else slot1, bn, bn)

            def grp_body(g, carry2):
                vvec = vals_e[slot, b, pl.ds(g * 16, 16)]
                for j in range(16):
                    r = g * 16 + j
                    v = jnp.broadcast_to(vvec[j], (16,))
                    for k in range(D // 16):
                        sl = pl.ds(k * 16, 16)
                        gbuf[b][r, sl] = gbuf[b][r, sl] * v
                return carry2
            lax.fori_loop(0, CHUNK // 16, grp_body, 0)

            sissue(slot, b, b)
        return carry
    lax.fori_loop(0, NGROUPS, group_body, 0)

    last_slot = (NGROUPS - 1) % 3
    swait(last_slot, NBUF - 2, NBUF - 2)
    swait(last_slot, NBUF - 1, NBUF - 1)
    plsc.subcore_barrier()
    pltpu.sync_copy(shared.at[pl.ds(s * ROWS_PER_TILE, ROWS_PER_TILE)],
                    out_hbm.at[c, pl.ds(s * ROWS_PER_TILE, ROWS_PER_TILE)])


_sc_propagate = functools.partial(
    pl.kernel,
    out_type=jax.ShapeDtypeStruct((NC, NP, D), jnp.float32),
    mesh=_mesh,
    scratch_types=[
        pltpu.VMEM((3, NBUF, CHUNK), jnp.int32),
        pltpu.VMEM((3, NBUF, CHUNK), jnp.int32),
        pltpu.VMEM((3, NBUF, CHUNK), jnp.float32),
    ] + [pltpu.VMEM((CHUNK, D), jnp.float32)] * NBUF + [
        pltpu.VMEM_SHARED((NP, D), jnp.float32),
    ] + [pltpu.SemaphoreType.DMA] * (2 * NBUF + 1),
)(_prop_body)


def _gather_body(table_hbm, idx3_hbm, out_hbm, idx_v, buf_v, sem):
    c = lax.axis_index("c")
    s = lax.axis_index("s")
    wid = c * NS + s
    pltpu.sync_copy(idx3_hbm.at[wid], idx_v)
    for j in range(4):
        pltpu.async_copy(table_hbm.at[idx_v.at[j]], buf_v, sem).wait()
        pltpu.sync_copy(
            buf_v, out_hbm.at[pl.ds(wid * 4 * GCHUNK + j * GCHUNK, GCHUNK)])


_sc_gather = functools.partial(
    pl.kernel,
    out_type=jax.ShapeDtypeStruct((4 * BATCH, D), jnp.float32),
    mesh=_mesh,
    scratch_types=[
        pltpu.VMEM((4, GCHUNK), jnp.int32),
        pltpu.VMEM((GCHUNK, D), jnp.float32),
        pltpu.SemaphoreType.DMA,
    ],
)(_gather_body)


def _leaky_norm(p):
    p = jnp.where(p >= 0, p, 0.1 * p)
    n = jnp.sqrt(jnp.sum(p * p, axis=-1, keepdims=True))
    return p / jnp.maximum(n, 1e-12)


def _norm_body(x_ref, o_ref):
    o_ref[...] = _leaky_norm(x_ref[...])


def _tc_norm(x, rows_per_block):
    n_rows = x.shape[0]
    grid = n_rows // rows_per_block
    return pl.pallas_call(
        _norm_body,
        grid=(grid,),
        in_specs=[pl.BlockSpec((rows_per_block, D), lambda i: (i, 0))],
        out_specs=pl.BlockSpec((rows_per_block, D), lambda i: (i, 0)),
        out_shape=jax.ShapeDtypeStruct((n_rows, D), jnp.float32),
    )(x)


def _combine_body(p_ref, acc_ref, o_ref, acco_ref):
    nrm = _leaky_norm(p_ref[0] + p_ref[1])
    o_ref[...] = nrm
    acco_ref[...] = acc_ref[...] + nrm


def _tc_combine(parts, acc):
    rb = 1024
    grid = NP // rb
    return pl.pallas_call(
        _combine_body,
        grid=(grid,),
        in_specs=[
            pl.BlockSpec((NC, rb, D), lambda i: (0, i, 0)),
            pl.BlockSpec((rb, D), lambda i: (i, 0)),
        ],
        out_specs=[
            pl.BlockSpec((rb, D), lambda i: (i, 0)),
            pl.BlockSpec((rb, D), lambda i: (i, 0)),
        ],
        out_shape=[
            jax.ShapeDtypeStruct((NP, D), jnp.float32),
            jax.ShapeDtypeStruct((NP, D), jnp.float32),
        ],
    )(parts, acc)


def _dots_body(g_ref, a_ref, w_ref, s_ref):
    scale = 1.0 / ((N_LAYERS + 1) * (N_LAYERS + 1))
    u = g_ref[0]
    a_ref[...] = jax.nn.sigmoid(jnp.sum(u * g_ref[1], axis=-1) * scale)
    w_ref[...] = jax.nn.sigmoid(jnp.sum(u * g_ref[2], axis=-1) * scale)
    s_ref[...] = jax.nn.sigmoid(jnp.sum(u * g_ref[3], axis=-1) * scale)


def _tc_dots(g4):
    bb = 1024
    grid = BATCH // bb
    out = jax.ShapeDtypeStruct((BATCH,), jnp.float32)
    return pl.pallas_call(
        _dots_body,
        grid=(grid,),
        in_specs=[pl.BlockSpec((4, bb, D), lambda i: (0, i, 0))],
        out_specs=[pl.BlockSpec((bb,), lambda i: (i,))] * 3,
        out_shape=[out, out, out],
    )(g4)


def kernel(users, adjacent_items, weak_items, strong_items, edge_index,
           edge_values, user_preference, item_preference):
    raw = jnp.concatenate(
        [user_preference, item_preference,
         jnp.zeros((NP - N_NODES, D), jnp.float32)], axis=0)
    pref = _tc_norm(raw, 1024)
    acc = pref

    rows = edge_index[0]
    cols = edge_index[1]
    pad = NW * EDGES_PER_TILE - N_EDGES
    eshape = (NW, NGROUPS, NBUF, CHUNK)
    spread = (jnp.arange(pad, dtype=jnp.int32) * 13) % N_NODES
    rows4 = jnp.concatenate([rows, spread])
    rows4 = rows4.reshape(eshape)
    cols4 = jnp.concatenate([cols, spread])
    cols4 = cols4.reshape(eshape)
    vals4 = jnp.concatenate([edge_values, jnp.zeros((pad,), jnp.float32)])
    vals4 = vals4.reshape(eshape)
    zeros = jnp.zeros((NP, D), jnp.float32)

    for _ in range(N_LAYERS):
        parts = _sc_propagate(pref, cols4, rows4, vals4, zeros)
        pref, acc = _tc_combine(parts, acc)

    idx = jnp.concatenate([
        users,
        adjacent_items + NUM_USERS,
        weak_items + NUM_USERS,
        strong_items + NUM_USERS,
    ]).reshape(NW, 4, GCHUNK)
    g = _sc_gather(acc, idx)
    return _tc_dots(g.reshape(4, BATCH, D))
